# Initial kernel scaffold; baseline (speedup 1.0000x reference)
import jax
import jax.numpy as jnp
from jax.experimental import pallas as pl


def kernel(x_entity, x_attribute, params, edge_index_ee, edge_index_ae, edge_index_ea):
    raise NotImplementedError("write your pallas kernel here")


# speedup vs baseline: 3.2066x; 3.2066x over previous
import functools

import jax
import jax.numpy as jnp
from jax import lax
from jax.experimental import pallas as pl
from jax.experimental.pallas import tpu as pltpu
from jax.experimental.pallas import tpu_sc as plsc

_NE = 50000
_NA = 10000
_D = 128

_EP_EE = 262144
_EP_AE = 131072
_EP_EA = 131072

_LC_E = 50176
_LC_A = 10240

_NB = 3
_LAG = 2


def _pad_edges(ei, e_pad, dummy_dst):
    src = jnp.pad(ei[0], (0, e_pad - ei.shape[1]))
    dst = jnp.pad(ei[1], (0, e_pad - ei.shape[1]), constant_values=dummy_dst)
    return src.reshape(e_pad // 128, 128), dst.reshape(e_pad // 128, 128)


def _counts_body(dee_hbm, dae_hbm, dea_hbm, oee, oae, oea,
                 cee, cae, cea, dchunk):
    c = lax.axis_index("c")
    s = lax.axis_index("s")
    tid = c * 16 + s

    ones = jnp.ones((16,), jnp.float32)
    zeros = jnp.zeros((16,), jnp.float32)

    def _z(ref, n):
        def body(i, _):
            ref[pl.ds(i * 16, 16)] = zeros
            return 0
        lax.fori_loop(0, n // 16, body, 0)
    _z(cee, _LC_E)
    _z(cae, _LC_A)
    _z(cea, _LC_A)

    def _hist(dst_hbm, cnt_ref, rows_per_tile):
        def ic_body(ic, _):
            r0 = tid * rows_per_tile + ic * 16
            pltpu.sync_copy(dst_hbm.at[pl.ds(r0, 16), :], dchunk)

            def row_body(j, _):
                for k in range(8):
                    d16 = dchunk[j, pl.ds(k * 16, 16)]
                    plsc.addupdate_scatter(cnt_ref, [d16], ones)
                return 0
            lax.fori_loop(0, 16, row_body, 0)
            return 0
        lax.fori_loop(0, rows_per_tile // 16, ic_body, 0)

    _hist(dee_hbm, cee, _EP_EE // 128 // 32)
    _hist(dae_hbm, cae, _EP_AE // 128 // 32)
    _hist(dea_hbm, cea, _EP_EA // 128 // 32)

    pltpu.sync_copy(cee, oee.at[tid])
    pltpu.sync_copy(cae, oae.at[tid])
    pltpu.sync_copy(cea, oea.at[tid])


def _sc_counts(dst2_ee, dst2_ae, dst2_ea):
    mesh = plsc.VectorSubcoreMesh(core_axis_name="c", subcore_axis_name="s")
    out_type = (
        jax.ShapeDtypeStruct((32, _LC_E), jnp.float32),
        jax.ShapeDtypeStruct((32, _LC_A), jnp.float32),
        jax.ShapeDtypeStruct((32, _LC_A), jnp.float32),
    )
    scratch = [
        pltpu.VMEM((_LC_E,), jnp.float32),
        pltpu.VMEM((_LC_A,), jnp.float32),
        pltpu.VMEM((_LC_A,), jnp.float32),
        pltpu.VMEM((16, 128), jnp.int32),
    ]
    f = pl.kernel(_counts_body, out_type=out_type, mesh=mesh,
                  scratch_types=scratch,
                  compiler_params=pltpu.CompilerParams(
                      use_tc_tiling_on_sc=False, needs_layout_passes=False))
    return f(dst2_ee, dst2_ae, dst2_ea)


def _inv_body(p_ref, o_ref):
    s = jnp.sum(p_ref[...], axis=0)
    o_ref[...] = 1.0 / jnp.maximum(s, 1.0)


def _tc_invcnt(p, n):
    rows = p.shape[1] // 128
    out = pl.pallas_call(
        _inv_body,
        out_shape=jax.ShapeDtypeStruct((rows, 128), jnp.float32),
    )(p.reshape(32, rows, 128))
    return out.reshape(rows * 128, 1)[:n]


def _segsum_body(ytab, src2, dst2, out, acc, sbuf, dbuf, rows, zbuf,
                 gsems, ssems, *, n_dst, dc, n_pass, n_ic):
    c = lax.axis_index("c")
    s = lax.axis_index("s")
    stripe = n_dst // 16
    zr = zbuf.shape[0]
    n_zc = stripe // zr

    def zb(i, _):
        for k in range(dc // 16):
            zbuf[i, pl.ds(k * 16, 16)] = jnp.zeros((16,), jnp.float32)
        return 0
    lax.fori_loop(0, zr, zb, 0)

    def zero_acc():
        for z in range(n_zc):
            pltpu.sync_copy(zbuf, acc.at[pl.ds(s * stripe + z * zr, zr), :])

    zero_acc()
    plsc.subcore_barrier()

    for p in range(n_pass):
        q = c * n_pass + p
        tbl = ytab.at[q]

        def ic_body(ic, _):
            r0 = s * (n_ic * 16) + ic * 16
            pltpu.sync_copy(src2.at[pl.ds(r0, 16), :], sbuf)
            pltpu.sync_copy(dst2.at[pl.ds(r0, 16), :], dbuf)
            gd = [None] * _NB
            sd = [None] * _NB
            for t in range(16 + _LAG):
                if t < 16:
                    b = t % _NB
                    if t >= _NB:
                        sd[b].wait()
                    gd[b] = pltpu.async_copy(
                        tbl.at[sbuf.at[t]],
                        rows.at[pl.ds(b * 128, 128), :], gsems[b])
                if t >= _LAG:
                    i = t - _LAG
                    bi = i % _NB
                    gd[bi].wait()
                    sd[bi] = pltpu.async_copy(
                        rows.at[pl.ds(bi * 128, 128), :],
                        acc.at[dbuf.at[i]], ssems[bi], add=True)
            for i in range(16 - _NB, 16):
                sd[i % _NB].wait()
            return 0

        lax.fori_loop(0, n_ic, ic_body, 0)
        plsc.subcore_barrier()
        pltpu.sync_copy(acc.at[pl.ds(s * stripe, stripe), :],
                        out.at[q, pl.ds(s * stripe, stripe), :])
        if p + 1 < n_pass:
            zero_acc()
            plsc.subcore_barrier()


def _sc_segsum(ytab, src2, dst2, n_dst, n_pass):
    nch, n_src, dc = ytab.shape
    e_pad = src2.shape[0] * 128
    n_ic = e_pad // 128 // 16 // 16
    stripe = n_dst // 16
    zr = 125 if stripe % 125 == 0 else stripe
    mesh = plsc.VectorSubcoreMesh(core_axis_name="c", subcore_axis_name="s")
    body = functools.partial(_segsum_body, n_dst=n_dst, dc=dc,
                             n_pass=n_pass, n_ic=n_ic)
    scratch = [
        pltpu.VMEM_SHARED((n_dst + 8, dc), jnp.float32),
        pltpu.VMEM((16, 128), jnp.int32),
        pltpu.VMEM((16, 128), jnp.int32),
        pltpu.VMEM((_NB * 128, dc), jnp.float32),
        pltpu.VMEM((zr, dc), jnp.float32),
        [pltpu.SemaphoreType.DMA] * _NB,
        [pltpu.SemaphoreType.DMA] * _NB,
    ]
    f = pl.kernel(body, out_type=jax.ShapeDtypeStruct((nch, n_dst, dc),
                                                      jnp.float32),
                  mesh=mesh, scratch_types=scratch,
                  compiler_params=pltpu.CompilerParams(
                      use_tc_tiling_on_sc=False, needs_layout_passes=False))
    return f(ytab, src2, dst2)


_BN = 1000


def _split_writes(cat, out_refs, specs):
    col = 0
    for o_ref, (nc, cc) in zip(out_refs, specs):
        for p in range(nc):
            o_ref[p] = cat[:, col:col + cc]
            col += cc


def _proj_cat_body(x_ref, w1_ref, b1_ref, wc_ref, *out_refs, specs):
    h = jnp.maximum(
        jnp.dot(x_ref[...], w1_ref[...],
                preferred_element_type=jnp.float32) + b1_ref[...], 0.0)
    cat = jnp.dot(h, wc_ref[...], preferred_element_type=jnp.float32)
    _split_writes(cat, out_refs, specs)


def _tc_proj_cat(x, w1, b1, wc, specs):
    n = x.shape[0]
    kcols = wc.shape[1]
    out_shape = [jax.ShapeDtypeStruct((nc, n, cc), jnp.float32)
                 for nc, cc in specs]
    out_specs = [pl.BlockSpec((nc, _BN, cc), lambda i: (0, i, 0))
                 for nc, cc in specs]
    return pl.pallas_call(
        functools.partial(_proj_cat_body, specs=specs),
        grid=(n // _BN,),
        in_specs=[
            pl.BlockSpec((_BN, _D), lambda i: (i, 0)),
            pl.BlockSpec((_D, _D), lambda i: (0, 0)),
            pl.BlockSpec((1, _D), lambda i: (0, 0)),
            pl.BlockSpec((_D, kcols), lambda i: (0, 0)),
        ],
        out_specs=out_specs,
        out_shape=out_shape,
    )(x, w1, b1, wc)


def _combine_cat_body(*refs, specs, has_ee, nb_a, final):
    it = iter(refs)
    m = jnp.zeros((_BN, _D), jnp.float32)
    if has_ee:
        see_ref = next(it)
        cee_ref = next(it)
        see = jnp.concatenate([see_ref[p] for p in range(4)], axis=-1)
        m = m + see * cee_ref[...]
    sa_ref = next(it)
    ca_ref = next(it)
    sa = jnp.concatenate([sa_ref[0], sa_ref[1]], axis=-1)
    ma = sa * ca_ref[...]
    if has_ee:
        i = pl.program_id(0)
        ma = jnp.where(i < nb_a, ma, 0.0)
    m = m + ma
    r_ref = next(it)
    b_ref = next(it)
    wc_ref = next(it)
    h = jnp.maximum(m + r_ref[...] + b_ref[...], 0.0)
    cat = jnp.dot(h, wc_ref[...], preferred_element_type=jnp.float32)
    rest = list(it)
    if final:
        cat = cat + rest[0][...]
        rest = rest[1:]
    _split_writes(cat, rest, specs)


def _tc_combine_cat(s_ee, cnt_ee, s_a, cnt_a, r, b, wc, bo, specs):
    if r.ndim == 3:
        r = r[0]
    n = r.shape[0]
    has_ee = s_ee is not None
    nb_a = s_a.shape[1] // _BN
    kcols = wc.shape[1]
    in_specs = []
    args = []
    if has_ee:
        in_specs += [
            pl.BlockSpec((4, _BN, 32), lambda i: (0, i, 0)),
            pl.BlockSpec((_BN, 1), lambda i: (i, 0)),
        ]
        args += [s_ee, cnt_ee]
    cl = nb_a - 1
    in_specs += [
        pl.BlockSpec((2, _BN, 64),
                     lambda i, cl=cl: (0, jnp.minimum(i, cl), 0)),
        pl.BlockSpec((_BN, 1), lambda i, cl=cl: (jnp.minimum(i, cl), 0)),
        pl.BlockSpec((_BN, _D), lambda i: (i, 0)),
        pl.BlockSpec((1, _D), lambda i: (0, 0)),
        pl.BlockSpec((_D, kcols), lambda i: (0, 0)),
    ]
    args += [s_a, cnt_a, r, b, wc]
    if bo is not None:
        in_specs.append(pl.BlockSpec((1, kcols), lambda i: (0, 0)))
        args.append(bo)
    out_shape = [jax.ShapeDtypeStruct((nc, n, cc), jnp.float32)
                 for nc, cc in specs]
    out_specs = [pl.BlockSpec((nc, _BN, cc), lambda i: (0, i, 0))
                 for nc, cc in specs]
    body = functools.partial(_combine_cat_body, specs=specs, has_ee=has_ee,
                             nb_a=nb_a, final=bo is not None)
    return pl.pallas_call(
        body, grid=(n // _BN,), in_specs=in_specs, out_specs=out_specs,
        out_shape=out_shape,
    )(*args)


_E_SPECS = [(4, 32), (2, 64), (1, 128)]
_A_SPECS = [(2, 64), (1, 128)]
_Z_SPECS = [(1, 128)]


def kernel(x_entity, x_attribute, params, edge_index_ee, edge_index_ae,
           edge_index_ea):
    We, be = params['lin']['entity']
    Wa, ba = params['lin']['attribute']
    Woe, boe = params['out']['entity']
    Woa, boa = params['out']['attribute']
    convs = params['convs']

    src_ee, dst_ee = _pad_edges(edge_index_ee, _EP_EE, _NE)
    src_ae, dst_ae = _pad_edges(edge_index_ae, _EP_AE, _NA)
    src_ea, dst_ea = _pad_edges(edge_index_ea, _EP_EA, _NA)

    cnt_ee_p, cnt_ae_p, cnt_ea_p = _sc_counts(dst_ee, dst_ae, dst_ea)
    cnt_ee_p = _tc_invcnt(cnt_ee_p, _NE)
    cnt_ae_p = _tc_invcnt(cnt_ae_p, _NA)
    cnt_ea_p = _tc_invcnt(cnt_ea_p, _NA)

    def ewc(layer):
        Wl_ee, _, Wr_ee = layer['ee']
        Wl_ea, _, _ = layer['ea']
        _, _, Wr_ae = layer['ae']
        return jnp.concatenate([Wl_ee, Wl_ea, Wr_ee + Wr_ae], axis=1)

    def awc(layer):
        Wl_ae, _, _ = layer['ae']
        _, _, Wr_ea = layer['ea']
        return jnp.concatenate([Wl_ae, Wr_ea], axis=1)

    def ebias(layer):
        return (layer['ee'][1] + layer['ae'][1]).reshape(1, _D)

    def abias(layer):
        return layer['ea'][1].reshape(1, _D)

    yee, yea, r_e = _tc_proj_cat(x_entity, We, be.reshape(1, _D),
                                 ewc(convs[0]), _E_SPECS)
    yae, r_a = _tc_proj_cat(x_attribute, Wa, ba.reshape(1, _D),
                            awc(convs[0]), _A_SPECS)

    for li in range(2):
        s_ee = _sc_segsum(yee, src_ee, dst_ee, _NE, 2)
        s_ea = _sc_segsum(yea, src_ea, dst_ea, _NA, 1)
        s_ae = _sc_segsum(yae, src_ae, dst_ae, _NA, 1)
        if li == 0:
            yee, yea, r_e = _tc_combine_cat(
                s_ee, cnt_ee_p, s_ae, cnt_ae_p, r_e, ebias(convs[0]),
                ewc(convs[1]), None, _E_SPECS)
            yae, r_a = _tc_combine_cat(
                None, None, s_ea, cnt_ea_p, r_a, abias(convs[0]),
                awc(convs[1]), None, _A_SPECS)
        else:
            (z_e,) = _tc_combine_cat(
                s_ee, cnt_ee_p, s_ae, cnt_ae_p, r_e, ebias(convs[1]),
                Woe, boe.reshape(1, _D), _Z_SPECS)
            (z_a,) = _tc_combine_cat(
                None, None, s_ea, cnt_ea_p, r_a, abias(convs[1]),
                Woa, boa.reshape(1, _D), _Z_SPECS)

    return (z_e[0], z_a[0])
